# baseline (speedup 1.0000x reference)
import functools

import jax
import jax.numpy as jnp
from jax import lax
from jax.experimental import pallas as pl
from jax.experimental.pallas import tpu as pltpu
from jax.experimental.pallas import tpu_sc as plsc

B = 512; K = 3; H = 128; DB = 16; NH = 8; DH = 16; OUT = 1000
NM = 24576; EM = 196608; NA = 24576; EA = 98304

_PREC = lax.Precision.HIGHEST


def _dot(a, b):
    return jnp.dot(a, b, precision=_PREC, preferred_element_type=jnp.float32)


def _enc_body(x_ref, w_ref, b_ref, o_ref):
    o_ref[...] = jnp.maximum(_dot(x_ref[...], w_ref[...]) + b_ref[...], 0.0)


def _node_encode(x, w, b):
    M = x.shape[0]
    RB = 2048
    return pl.pallas_call(
        _enc_body,
        grid=(M // RB,),
        in_specs=[
            pl.BlockSpec((RB, x.shape[1]), lambda i: (i, 0)),
            pl.BlockSpec(w.shape, lambda i: (0, 0)),
            pl.BlockSpec((1, H), lambda i: (0, 0)),
        ],
        out_specs=pl.BlockSpec((RB, H), lambda i: (i, 0)),
        out_shape=jax.ShapeDtypeStruct((M, H), jnp.float32),
    )(x, w, b.reshape(1, H))


def _t_body(h_ref, w_ref, lo_ref, hi_ref):
    y = _dot(h_ref[...], w_ref[...])
    lo_ref[...] = y[:, :64]
    hi_ref[...] = y[:, 64:]


def _t_halves(h, w):
    M = h.shape[0]
    RB = 2048
    return pl.pallas_call(
        _t_body,
        grid=(M // RB,),
        in_specs=[
            pl.BlockSpec((RB, H), lambda i: (i, 0)),
            pl.BlockSpec((H, H), lambda i: (0, 0)),
        ],
        out_specs=[pl.BlockSpec((RB, 64), lambda i: (i, 0))] * 2,
        out_shape=[jax.ShapeDtypeStruct((M, 64), jnp.float32)] * 2,
    )(h, w)


def _u_body(ea_ref, w_ref, b_ref, lo_ref, hi_ref):
    y = _dot(ea_ref[...], w_ref[...]) + b_ref[...]
    lo_ref[...] = y[:, :64]
    hi_ref[...] = y[:, 64:]


def _u_halves(ea, w, b):
    E = ea.shape[0]
    RB = 4096
    return pl.pallas_call(
        _u_body,
        grid=(E // RB,),
        in_specs=[
            pl.BlockSpec((RB, DB), lambda i: (i, 0)),
            pl.BlockSpec((DB, H), lambda i: (0, 0)),
            pl.BlockSpec((1, H), lambda i: (0, 0)),
        ],
        out_specs=[pl.BlockSpec((RB, 64), lambda i: (i, 0))] * 2,
        out_shape=[jax.ShapeDtypeStruct((E, 64), jnp.float32)] * 2,
    )(ea, w, b.reshape(1, H))


def _upd_body(h_ref, alo_ref, ahi_ref, w1_ref, w2a_ref, w2b_ref, b_ref, o_ref):
    y = (_dot(h_ref[...], w1_ref[...]) + _dot(alo_ref[...], w2a_ref[...])
         + _dot(ahi_ref[...], w2b_ref[...]) + b_ref[...])
    o_ref[...] = jnp.maximum(y, 0.0)


def _update(h, agg_lo, agg_hi, wu, bu):
    M = h.shape[0]
    RB = 2048
    return pl.pallas_call(
        _upd_body,
        grid=(M // RB,),
        in_specs=[
            pl.BlockSpec((RB, H), lambda i: (i, 0)),
            pl.BlockSpec((RB, 64), lambda i: (i, 0)),
            pl.BlockSpec((RB, 64), lambda i: (i, 0)),
            pl.BlockSpec((H, H), lambda i: (0, 0)),
            pl.BlockSpec((64, H), lambda i: (0, 0)),
            pl.BlockSpec((64, H), lambda i: (0, 0)),
            pl.BlockSpec((1, H), lambda i: (0, 0)),
        ],
        out_specs=pl.BlockSpec((RB, H), lambda i: (i, 0)),
        out_shape=jax.ShapeDtypeStruct((M, H), jnp.float32),
    )(h, agg_lo, agg_hi, wu[:H], wu[H:H + 64], wu[H + 64:], bu.reshape(1, H))


def _updw_body(h_ref, alo_ref, ahi_ref, w1_ref, w2a_ref, w2b_ref, b_ref,
               w_ref, lo_ref, hi_ref):
    y = (_dot(h_ref[...], w1_ref[...]) + _dot(alo_ref[...], w2a_ref[...])
         + _dot(ahi_ref[...], w2b_ref[...]) + b_ref[...])
    hw = jnp.maximum(y, 0.0) * w_ref[...]
    lo_ref[...] = hw[:, :64]
    hi_ref[...] = hw[:, 64:]


def _update_weighted(h, agg_lo, agg_hi, wu, bu, fw):
    M = h.shape[0]
    RB = 2048
    return pl.pallas_call(
        _updw_body,
        grid=(M // RB,),
        in_specs=[
            pl.BlockSpec((RB, H), lambda i: (i, 0)),
            pl.BlockSpec((RB, 64), lambda i: (i, 0)),
            pl.BlockSpec((RB, 64), lambda i: (i, 0)),
            pl.BlockSpec((H, H), lambda i: (0, 0)),
            pl.BlockSpec((64, H), lambda i: (0, 0)),
            pl.BlockSpec((64, H), lambda i: (0, 0)),
            pl.BlockSpec((1, H), lambda i: (0, 0)),
            pl.BlockSpec((RB, 1), lambda i: (i, 0)),
        ],
        out_specs=[pl.BlockSpec((RB, 64), lambda i: (i, 0))] * 2,
        out_shape=[jax.ShapeDtypeStruct((M, 64), jnp.float32)] * 2,
    )(h, agg_lo, agg_hi, wu[:H], wu[H:H + 64], wu[H + 64:], bu.reshape(1, H),
      fw.reshape(M, 1))


def _edge_aggregate(t_lo, t_hi, u_lo, u_hi, src, dst, n):
    agg_lo = jax.ops.segment_sum(jnp.maximum(t_lo[src] + u_lo, 0.0), dst,
                                 num_segments=n)
    agg_hi = jax.ops.segment_sum(jnp.maximum(t_hi[src] + u_hi, 0.0), dst,
                                 num_segments=n)
    return agg_lo, agg_hi


def _pool(hw_lo, hw_hi, seg, s):
    return (jax.ops.segment_sum(hw_lo, seg, num_segments=s),
            jax.ops.segment_sum(hw_hi, seg, num_segments=s))


def _gnn_embed(x, ei, ea, fw, batch, gp, num_segments):
    n = x.shape[0]
    src, dst = ei[0], ei[1]
    h = _node_encode(x, gp['Win'], gp['bin'])
    nl = len(gp['layers'])
    for li, lp in enumerate(gp['layers']):
        t_lo, t_hi = _t_halves(h, lp['Wm'][:H])
        u_lo, u_hi = _u_halves(ea, lp['Wm'][H:], lp['bm'])
        agg_lo, agg_hi = _edge_aggregate(t_lo, t_hi, u_lo, u_hi, src, dst, n)
        if li < nl - 1:
            h = _update(h, agg_lo, agg_hi, lp['Wu'], lp['bu'])
        else:
            hw_lo, hw_hi = _update_weighted(h, agg_lo, agg_hi, lp['Wu'],
                                            lp['bu'], fw)
    return _pool(hw_lo, hw_hi, batch, num_segments)


def _ln_in(x):
    m = jnp.mean(x, axis=-1, keepdims=True)
    v = jnp.mean((x - m) * (x - m), axis=-1, keepdims=True)
    return (x - m) * lax.rsqrt(v + 1e-5)


def _softmax3(s0, s1, s2):
    m = jnp.maximum(jnp.maximum(s0, s1), s2)
    e0 = jnp.exp(s0 - m); e1 = jnp.exp(s1 - m); e2 = jnp.exp(s2 - m)
    d = e0 + e1 + e2
    return e0 / d, e1 / d, e2 / d


def _head_mask():
    d = lax.broadcasted_iota(jnp.int32, (H, NH), 0)
    hh = lax.broadcasted_iota(jnp.int32, (H, NH), 1)
    fwd = (d // DH == hh).astype(jnp.float32)
    d2 = lax.broadcasted_iota(jnp.int32, (NH, H), 1)
    h2 = lax.broadcasted_iota(jnp.int32, (NH, H), 0)
    bwd = (d2 // DH == h2).astype(jnp.float32)
    return fwd, bwd


def _sa_layer(a, wq, wk, wv, wo, w1, w2, mh):
    mhf, mhb = mh
    q = [_dot(x, wq) for x in a]
    k = [_dot(x, wk) for x in a]
    v = [_dot(x, wv) for x in a]
    out = []
    for i in range(3):
        s = [_dot(q[i] * k[j], mhf) * 0.25 for j in range(3)]
        p = _softmax3(*s)
        o = sum(_dot(p[j], mhb) * v[j] for j in range(3))
        x = _ln_in(a[i] + _dot(o, wo))
        x = _ln_in(x + _dot(jnp.maximum(_dot(x, w1), 0.0), w2))
        out.append(x)
    return out


def _ca_layer(y, ap, wq, wk, wv, wo, w1, w2, mh):
    mhf, mhb = mh
    q = _dot(y, wq)
    k = [_dot(x, wk) for x in ap]
    v = [_dot(x, wv) for x in ap]
    s = [_dot(q * k[j], mhf) * 0.25 for j in range(3)]
    p = _softmax3(*s)
    o = sum(_dot(p[j], mhb) * v[j] for j in range(3))
    y = _ln_in(y + _dot(o, wo))
    y = _ln_in(y + _dot(jnp.maximum(_dot(y, w1), 0.0), w2))
    return y


def _prelu(x, a):
    return jnp.where(x >= 0, x, a * x)


def _head_body(*refs):
    (main_lo, main_hi,
     e10, e11, e12, e13, e14, e15,
     e20, e21, e22, e23, e24, e25,
     fw1, fb1, fa1, fw2, fb2, fa2,
     cw1, cb1, ca_, cw2, cb2) = [r[...] for r in refs[:25]]
    attn = [r[...] for r in refs[25:25 + 48]]
    o_ref = refs[-1]
    mh = _head_mask()
    main = jnp.concatenate([main_lo, main_hi], axis=-1)

    def branch(embs, fw, fb, fa, aw, off):
        rep_t = _dot(main, fw[H:])
        a = []
        for kk in range(3):
            ek = jnp.concatenate([embs[2 * kk], embs[2 * kk + 1]], axis=-1)
            a.append(_prelu(_dot(ek, fw[:H]) + rep_t + fb, fa))
        for l in range(2):
            a = _sa_layer(a, *aw[off + 6 * l: off + 6 * l + 6], mh)
        y = main
        for l in range(2):
            y = _ca_layer(y, a, *aw[off + 12 + 6 * l: off + 18 + 6 * l], mh)
        return y

    f1 = branch([e10, e11, e12, e13, e14, e15], fw1, fb1, fa1, attn, 0)
    f2 = branch([e20, e21, e22, e23, e24, e25], fw2, fb2, fa2, attn, 24)
    z = jnp.concatenate([main, f1, f2], axis=-1)
    z = _prelu(_dot(z, cw1) + cb1, ca_)
    logit = _dot(z, cw2) + cb2
    o_ref[...] = 1.0 / (1.0 + jnp.exp(-logit))


def _head(main_lo, main_hi, p1_lo, p1_hi, p2_lo, p2_hi, params):
    embs1 = []
    embs2 = []
    for kk in range(K):
        embs1 += [p1_lo.reshape(B, K, 64)[:, kk], p1_hi.reshape(B, K, 64)[:, kk]]
        embs2 += [p2_lo.reshape(B, K, 64)[:, kk], p2_hi.reshape(B, K, 64)[:, kk]]
    cls = params['cls']
    w2p = jnp.pad(cls['W2'], ((0, 0), (0, 1024 - OUT)))
    b2p = jnp.pad(cls['b2'], (0, 1024 - OUT)).reshape(1, 1024)
    args = ([main_lo, main_hi] + embs1 + embs2 +
            [params['fuse1']['W'], params['fuse1']['b'].reshape(1, H),
             params['fuse1']['a'].reshape(1, 1),
             params['fuse2']['W'], params['fuse2']['b'].reshape(1, H),
             params['fuse2']['a'].reshape(1, 1),
             cls['W1'], cls['b1'].reshape(1, 3 * H), cls['a'].reshape(1, 1),
             w2p, b2p])
    for group in ('sa1', 'ca1', 'sa2', 'ca2'):
        for lp in params[group]:
            args += [lp['Wq'], lp['Wk'], lp['Wv'], lp['Wo'], lp['W1'], lp['W2']]
    out = pl.pallas_call(
        _head_body,
        out_shape=jax.ShapeDtypeStruct((B, 1024), jnp.float32),
    )(*args)
    return out[:, :OUT]


def kernel(main_x, main_edge_index, main_edge_attr, main_fc_weight, main_batch,
           add_x, add_edge_index, add_edge_attr, add_fc_weight, add_batch,
           add2_x, add2_edge_index, add2_edge_attr, add2_fc_weight, add2_batch,
           params):
    gp = params['gnn']
    main_lo, main_hi = _gnn_embed(main_x, main_edge_index, main_edge_attr,
                                  main_fc_weight, main_batch, gp, B)
    p1_lo, p1_hi = _gnn_embed(add_x, add_edge_index, add_edge_attr,
                              add_fc_weight, add_batch, gp, B * K)
    p2_lo, p2_hi = _gnn_embed(add2_x, add2_edge_index, add2_edge_attr,
                              add2_fc_weight, add2_batch, gp, B * K)
    return _head(main_lo, main_hi, p1_lo, p1_hi, p2_lo, p2_hi, params)


# speedup vs baseline: 2.7302x; 2.7302x over previous
import functools

import jax
import jax.numpy as jnp
from jax import lax
from jax.experimental import pallas as pl
from jax.experimental.pallas import tpu as pltpu
from jax.experimental.pallas import tpu_sc as plsc

B = 512; K = 3; H = 128; DB = 16; NH = 8; DH = 16; OUT = 1000
NM = 24576; EM = 196608; NA = 24576; EA = 98304

_PREC = lax.Precision.HIGHEST


def _dot(a, b):
    return jnp.dot(a, b, precision=_PREC, preferred_element_type=jnp.float32)


def _enc_body(x_ref, w_ref, b_ref, o_ref):
    o_ref[...] = jnp.maximum(_dot(x_ref[...], w_ref[...]) + b_ref[...], 0.0)


def _node_encode(x, w, b):
    M = x.shape[0]
    RB = 2048
    return pl.pallas_call(
        _enc_body,
        grid=(M // RB,),
        in_specs=[
            pl.BlockSpec((RB, x.shape[1]), lambda i: (i, 0)),
            pl.BlockSpec(w.shape, lambda i: (0, 0)),
            pl.BlockSpec((1, H), lambda i: (0, 0)),
        ],
        out_specs=pl.BlockSpec((RB, H), lambda i: (i, 0)),
        out_shape=jax.ShapeDtypeStruct((M, H), jnp.float32),
    )(x, w, b.reshape(1, H))


def _t_body(h_ref, w_ref, lo_ref, hi_ref):
    y = _dot(h_ref[...], w_ref[...])
    lo_ref[...] = y[:, :64]
    hi_ref[...] = y[:, 64:]


def _t_halves(h, w):
    M = h.shape[0]
    RB = 2048
    return pl.pallas_call(
        _t_body,
        grid=(M // RB,),
        in_specs=[
            pl.BlockSpec((RB, H), lambda i: (i, 0)),
            pl.BlockSpec((H, H), lambda i: (0, 0)),
        ],
        out_specs=[pl.BlockSpec((RB, 64), lambda i: (i, 0))] * 2,
        out_shape=[jax.ShapeDtypeStruct((M, 64), jnp.float32)] * 2,
    )(h, w)


def _u_body(ea_ref, w_ref, b_ref, lo_ref, hi_ref):
    y = _dot(ea_ref[...], w_ref[...]) + b_ref[...]
    lo_ref[...] = y[:, :64]
    hi_ref[...] = y[:, 64:]


def _u_halves(ea, w, b):
    E = ea.shape[0]
    RB = 4096
    return pl.pallas_call(
        _u_body,
        grid=(E // RB,),
        in_specs=[
            pl.BlockSpec((RB, DB), lambda i: (i, 0)),
            pl.BlockSpec((DB, H), lambda i: (0, 0)),
            pl.BlockSpec((1, H), lambda i: (0, 0)),
        ],
        out_specs=[pl.BlockSpec((RB, 64), lambda i: (i, 0))] * 2,
        out_shape=[jax.ShapeDtypeStruct((E, 64), jnp.float32)] * 2,
    )(ea, w, b.reshape(1, H))


def _upd_body(h_ref, alo_ref, ahi_ref, w1_ref, w2a_ref, w2b_ref, b_ref, o_ref):
    y = (_dot(h_ref[...], w1_ref[...]) + _dot(alo_ref[...], w2a_ref[...])
         + _dot(ahi_ref[...], w2b_ref[...]) + b_ref[...])
    o_ref[...] = jnp.maximum(y, 0.0)


def _update(h, agg_lo, agg_hi, wu, bu):
    M = h.shape[0]
    RB = 2048
    return pl.pallas_call(
        _upd_body,
        grid=(M // RB,),
        in_specs=[
            pl.BlockSpec((RB, H), lambda i: (i, 0)),
            pl.BlockSpec((RB, 64), lambda i: (i, 0)),
            pl.BlockSpec((RB, 64), lambda i: (i, 0)),
            pl.BlockSpec((H, H), lambda i: (0, 0)),
            pl.BlockSpec((64, H), lambda i: (0, 0)),
            pl.BlockSpec((64, H), lambda i: (0, 0)),
            pl.BlockSpec((1, H), lambda i: (0, 0)),
        ],
        out_specs=pl.BlockSpec((RB, H), lambda i: (i, 0)),
        out_shape=jax.ShapeDtypeStruct((M, H), jnp.float32),
    )(h, agg_lo, agg_hi, wu[:H], wu[H:H + 64], wu[H + 64:], bu.reshape(1, H))


def _updw_body(h_ref, alo_ref, ahi_ref, w1_ref, w2a_ref, w2b_ref, b_ref,
               w_ref, lo_ref, hi_ref):
    y = (_dot(h_ref[...], w1_ref[...]) + _dot(alo_ref[...], w2a_ref[...])
         + _dot(ahi_ref[...], w2b_ref[...]) + b_ref[...])
    hw = jnp.maximum(y, 0.0) * w_ref[...]
    lo_ref[...] = hw[:, :64]
    hi_ref[...] = hw[:, 64:]


def _update_weighted(h, agg_lo, agg_hi, wu, bu, fw):
    M = h.shape[0]
    RB = 2048
    return pl.pallas_call(
        _updw_body,
        grid=(M // RB,),
        in_specs=[
            pl.BlockSpec((RB, H), lambda i: (i, 0)),
            pl.BlockSpec((RB, 64), lambda i: (i, 0)),
            pl.BlockSpec((RB, 64), lambda i: (i, 0)),
            pl.BlockSpec((H, H), lambda i: (0, 0)),
            pl.BlockSpec((64, H), lambda i: (0, 0)),
            pl.BlockSpec((64, H), lambda i: (0, 0)),
            pl.BlockSpec((1, H), lambda i: (0, 0)),
            pl.BlockSpec((RB, 1), lambda i: (i, 0)),
        ],
        out_specs=[pl.BlockSpec((RB, 64), lambda i: (i, 0))] * 2,
        out_shape=[jax.ShapeDtypeStruct((M, 64), jnp.float32)] * 2,
    )(h, agg_lo, agg_hi, wu[:H], wu[H:H + 64], wu[H + 64:], bu.reshape(1, H),
      fw.reshape(M, 1))


_SC_MESH = plsc.VectorSubcoreMesh(core_axis_name="c", subcore_axis_name="s",
                                  num_cores=2, num_subcores=16)
_NT = 16
_CH = 128


def _edge_aggregate(t_lo, t_hi, u_lo, u_hi, src, dst, n):
    e = src.shape[0]
    per_tile = e // _NT
    n_chunks = per_tile // _CH
    rows_per_tile = n // _NT

    @functools.partial(
        pl.kernel,
        out_type=[jax.ShapeDtypeStruct((n, 64), jnp.float32)] * 2,
        mesh=_SC_MESH,
        scratch_types=[
            pltpu.VMEM((_CH,), jnp.int32),
            pltpu.VMEM((_CH,), jnp.int32),
            pltpu.VMEM((_CH, 64), jnp.float32),
            pltpu.VMEM((_CH, 64), jnp.float32),
            pltpu.VMEM_SHARED((n, 64), jnp.float32),
            pltpu.SemaphoreType.DMA,
        ],
        compiler_params=pltpu.CompilerParams(use_tc_tiling_on_sc=False),
    )
    def k(tlo, thi, ulo, uhi, srcr, dstr, zer, outlo, outhi,
          sidx, didx, trows, urows, acc, sem):
        c = lax.axis_index("c")
        s = lax.axis_index("s")
        r0 = s * rows_per_tile
        pltpu.sync_copy(zer.at[pl.ds(r0, rows_per_tile)],
                        acc.at[pl.ds(r0, rows_per_tile)])
        plsc.subcore_barrier()
        base = s * per_tile

        def body(g, carry):
            e0 = base + g * _CH
            pltpu.sync_copy(srcr.at[pl.ds(e0, _CH)], sidx)
            pltpu.sync_copy(dstr.at[pl.ds(e0, _CH)], didx)

            @pl.when(c == 0)
            def _():
                pltpu.sync_copy(ulo.at[pl.ds(e0, _CH)], urows)
                pltpu.async_copy(tlo.at[sidx], trows, sem).wait()

            @pl.when(c == 1)
            def _():
                pltpu.sync_copy(uhi.at[pl.ds(e0, _CH)], urows)
                pltpu.async_copy(thi.at[sidx], trows, sem).wait()

            def rowbody(r, rc):
                for j in range(4):
                    sl = pl.ds(j * 16, 16)
                    trows[r, sl] = jnp.maximum(trows[r, sl] + urows[r, sl],
                                               0.0)
                return rc
            lax.fori_loop(0, _CH, rowbody, 0, unroll=2)
            pltpu.sync_copy(trows, acc.at[didx], add=True)
            return carry

        lax.fori_loop(0, n_chunks, body, 0)
        plsc.subcore_barrier()

        @pl.when(c == 0)
        def _():
            pltpu.sync_copy(acc.at[pl.ds(r0, rows_per_tile)],
                            outlo.at[pl.ds(r0, rows_per_tile)])

        @pl.when(c == 1)
        def _():
            pltpu.sync_copy(acc.at[pl.ds(r0, rows_per_tile)],
                            outhi.at[pl.ds(r0, rows_per_tile)])

    zer = jnp.zeros((n, 64), jnp.float32)
    return k(t_lo, t_hi, u_lo, u_hi, src, dst, zer)


def _pool(hw_lo, hw_hi, seg, nseg):
    m = seg.shape[0]
    per_tile = m // _NT
    n_chunks = per_tile // _CH
    rows_per_tile = nseg // _NT

    @functools.partial(
        pl.kernel,
        out_type=[jax.ShapeDtypeStruct((nseg, 64), jnp.float32)] * 2,
        mesh=_SC_MESH,
        scratch_types=[
            pltpu.VMEM((_CH,), jnp.int32),
            pltpu.VMEM((_CH, 64), jnp.float32),
            pltpu.VMEM_SHARED((nseg, 64), jnp.float32),
        ],
        compiler_params=pltpu.CompilerParams(use_tc_tiling_on_sc=False),
    )
    def k(hwlo, hwhi, segr, zer, outlo, outhi, sidx, rows, acc):
        c = lax.axis_index("c")
        s = lax.axis_index("s")
        r0 = s * rows_per_tile
        pltpu.sync_copy(zer.at[pl.ds(r0, rows_per_tile)],
                        acc.at[pl.ds(r0, rows_per_tile)])
        plsc.subcore_barrier()
        base = s * per_tile

        def body(g, carry):
            e0 = base + g * _CH
            pltpu.sync_copy(segr.at[pl.ds(e0, _CH)], sidx)

            @pl.when(c == 0)
            def _():
                pltpu.sync_copy(hwlo.at[pl.ds(e0, _CH)], rows)

            @pl.when(c == 1)
            def _():
                pltpu.sync_copy(hwhi.at[pl.ds(e0, _CH)], rows)

            pltpu.sync_copy(rows, acc.at[sidx], add=True)
            return carry

        lax.fori_loop(0, n_chunks, body, 0)
        plsc.subcore_barrier()

        @pl.when(c == 0)
        def _():
            pltpu.sync_copy(acc.at[pl.ds(r0, rows_per_tile)],
                            outlo.at[pl.ds(r0, rows_per_tile)])

        @pl.when(c == 1)
        def _():
            pltpu.sync_copy(acc.at[pl.ds(r0, rows_per_tile)],
                            outhi.at[pl.ds(r0, rows_per_tile)])

    zer = jnp.zeros((nseg, 64), jnp.float32)
    return k(hw_lo, hw_hi, seg, zer)


def _gnn_embed(x, ei, ea, fw, batch, gp, num_segments):
    n = x.shape[0]
    src, dst = ei[0], ei[1]
    h = _node_encode(x, gp['Win'], gp['bin'])
    nl = len(gp['layers'])
    for li, lp in enumerate(gp['layers']):
        t_lo, t_hi = _t_halves(h, lp['Wm'][:H])
        u_lo, u_hi = _u_halves(ea, lp['Wm'][H:], lp['bm'])
        agg_lo, agg_hi = _edge_aggregate(t_lo, t_hi, u_lo, u_hi, src, dst, n)
        if li < nl - 1:
            h = _update(h, agg_lo, agg_hi, lp['Wu'], lp['bu'])
        else:
            hw_lo, hw_hi = _update_weighted(h, agg_lo, agg_hi, lp['Wu'],
                                            lp['bu'], fw)
    return _pool(hw_lo, hw_hi, batch, num_segments)


def _ln_in(x):
    m = jnp.mean(x, axis=-1, keepdims=True)
    v = jnp.mean((x - m) * (x - m), axis=-1, keepdims=True)
    return (x - m) * lax.rsqrt(v + 1e-5)


def _softmax3(s0, s1, s2):
    m = jnp.maximum(jnp.maximum(s0, s1), s2)
    e0 = jnp.exp(s0 - m); e1 = jnp.exp(s1 - m); e2 = jnp.exp(s2 - m)
    d = e0 + e1 + e2
    return e0 / d, e1 / d, e2 / d


def _head_mask():
    d = lax.broadcasted_iota(jnp.int32, (H, NH), 0)
    hh = lax.broadcasted_iota(jnp.int32, (H, NH), 1)
    fwd = (d // DH == hh).astype(jnp.float32)
    d2 = lax.broadcasted_iota(jnp.int32, (NH, H), 1)
    h2 = lax.broadcasted_iota(jnp.int32, (NH, H), 0)
    bwd = (d2 // DH == h2).astype(jnp.float32)
    return fwd, bwd


def _sa_layer(a, wq, wk, wv, wo, w1, w2, mh):
    mhf, mhb = mh
    q = [_dot(x, wq) for x in a]
    k = [_dot(x, wk) for x in a]
    v = [_dot(x, wv) for x in a]
    out = []
    for i in range(3):
        s = [_dot(q[i] * k[j], mhf) * 0.25 for j in range(3)]
        p = _softmax3(*s)
        o = sum(_dot(p[j], mhb) * v[j] for j in range(3))
        x = _ln_in(a[i] + _dot(o, wo))
        x = _ln_in(x + _dot(jnp.maximum(_dot(x, w1), 0.0), w2))
        out.append(x)
    return out


def _ca_layer(y, ap, wq, wk, wv, wo, w1, w2, mh):
    mhf, mhb = mh
    q = _dot(y, wq)
    k = [_dot(x, wk) for x in ap]
    v = [_dot(x, wv) for x in ap]
    s = [_dot(q * k[j], mhf) * 0.25 for j in range(3)]
    p = _softmax3(*s)
    o = sum(_dot(p[j], mhb) * v[j] for j in range(3))
    y = _ln_in(y + _dot(o, wo))
    y = _ln_in(y + _dot(jnp.maximum(_dot(y, w1), 0.0), w2))
    return y


def _prelu(x, a):
    return jnp.where(x >= 0, x, a * x)


def _head_body(*refs):
    (main_lo, main_hi,
     e10, e11, e12, e13, e14, e15,
     e20, e21, e22, e23, e24, e25,
     fw1, fb1, fa1, fw2, fb2, fa2,
     cw1, cb1, ca_, cw2, cb2) = [r[...] for r in refs[:25]]
    attn = [r[...] for r in refs[25:25 + 48]]
    o_ref = refs[-1]
    mh = _head_mask()
    main = jnp.concatenate([main_lo, main_hi], axis=-1)

    def branch(embs, fw, fb, fa, aw, off):
        rep_t = _dot(main, fw[H:])
        a = []
        for kk in range(3):
            ek = jnp.concatenate([embs[2 * kk], embs[2 * kk + 1]], axis=-1)
            a.append(_prelu(_dot(ek, fw[:H]) + rep_t + fb, fa))
        for l in range(2):
            a = _sa_layer(a, *aw[off + 6 * l: off + 6 * l + 6], mh)
        y = main
        for l in range(2):
            y = _ca_layer(y, a, *aw[off + 12 + 6 * l: off + 18 + 6 * l], mh)
        return y

    f1 = branch([e10, e11, e12, e13, e14, e15], fw1, fb1, fa1, attn, 0)
    f2 = branch([e20, e21, e22, e23, e24, e25], fw2, fb2, fa2, attn, 24)
    z = jnp.concatenate([main, f1, f2], axis=-1)
    z = _prelu(_dot(z, cw1) + cb1, ca_)
    logit = _dot(z, cw2) + cb2
    o_ref[...] = 1.0 / (1.0 + jnp.exp(-logit))


def _head(main_lo, main_hi, p1_lo, p1_hi, p2_lo, p2_hi, params):
    embs1 = []
    embs2 = []
    for kk in range(K):
        embs1 += [p1_lo.reshape(B, K, 64)[:, kk], p1_hi.reshape(B, K, 64)[:, kk]]
        embs2 += [p2_lo.reshape(B, K, 64)[:, kk], p2_hi.reshape(B, K, 64)[:, kk]]
    cls = params['cls']
    w2p = jnp.pad(cls['W2'], ((0, 0), (0, 1024 - OUT)))
    b2p = jnp.pad(cls['b2'], (0, 1024 - OUT)).reshape(1, 1024)
    args = ([main_lo, main_hi] + embs1 + embs2 +
            [params['fuse1']['W'], params['fuse1']['b'].reshape(1, H),
             params['fuse1']['a'].reshape(1, 1),
             params['fuse2']['W'], params['fuse2']['b'].reshape(1, H),
             params['fuse2']['a'].reshape(1, 1),
             cls['W1'], cls['b1'].reshape(1, 3 * H), cls['a'].reshape(1, 1),
             w2p, b2p])
    for group in ('sa1', 'ca1', 'sa2', 'ca2'):
        for lp in params[group]:
            args += [lp['Wq'], lp['Wk'], lp['Wv'], lp['Wo'], lp['W1'], lp['W2']]
    out = pl.pallas_call(
        _head_body,
        out_shape=jax.ShapeDtypeStruct((B, 1024), jnp.float32),
    )(*args)
    return out[:, :OUT]


def kernel(main_x, main_edge_index, main_edge_attr, main_fc_weight, main_batch,
           add_x, add_edge_index, add_edge_attr, add_fc_weight, add_batch,
           add2_x, add2_edge_index, add2_edge_attr, add2_fc_weight, add2_batch,
           params):
    gp = params['gnn']
    main_lo, main_hi = _gnn_embed(main_x, main_edge_index, main_edge_attr,
                                  main_fc_weight, main_batch, gp, B)
    p1_lo, p1_hi = _gnn_embed(add_x, add_edge_index, add_edge_attr,
                              add_fc_weight, add_batch, gp, B * K)
    p2_lo, p2_hi = _gnn_embed(add2_x, add2_edge_index, add2_edge_attr,
                              add2_fc_weight, add2_batch, gp, B * K)
    return _head(main_lo, main_hi, p1_lo, p1_hi, p2_lo, p2_hi, params)


# speedup vs baseline: 2.8004x; 1.0257x over previous
import functools

import jax
import jax.numpy as jnp
from jax import lax
from jax.experimental import pallas as pl
from jax.experimental.pallas import tpu as pltpu
from jax.experimental.pallas import tpu_sc as plsc

B = 512; K = 3; H = 128; DB = 16; NH = 8; DH = 16; OUT = 1000
NM = 24576; EM = 196608; NA = 24576; EA = 98304

_PREC = lax.Precision.HIGHEST


def _dot(a, b):
    return jnp.dot(a, b, precision=_PREC, preferred_element_type=jnp.float32)


def _enc_body(x_ref, w_ref, b_ref, o_ref):
    o_ref[...] = jnp.maximum(_dot(x_ref[...], w_ref[...]) + b_ref[...], 0.0)


def _node_encode(x, w, b):
    M = x.shape[0]
    RB = 2048
    return pl.pallas_call(
        _enc_body,
        grid=(M // RB,),
        in_specs=[
            pl.BlockSpec((RB, x.shape[1]), lambda i: (i, 0)),
            pl.BlockSpec(w.shape, lambda i: (0, 0)),
            pl.BlockSpec((1, H), lambda i: (0, 0)),
        ],
        out_specs=pl.BlockSpec((RB, H), lambda i: (i, 0)),
        out_shape=jax.ShapeDtypeStruct((M, H), jnp.float32),
    )(x, w, b.reshape(1, H))


def _t_body(h_ref, w_ref, o_ref):
    o_ref[...] = _dot(h_ref[...], w_ref[...])


def _t_flat(h, w):
    M = h.shape[0]
    RB = 2048
    ws = jnp.concatenate([w[:, :64], w[:, 64:]], axis=0)
    return pl.pallas_call(
        _t_body,
        grid=(2, M // RB),
        in_specs=[
            pl.BlockSpec((RB, H), lambda hh, i: (i, 0)),
            pl.BlockSpec((H, 64), lambda hh, i: (hh, 0)),
        ],
        out_specs=pl.BlockSpec((RB, 64), lambda hh, i: (hh * (M // RB) + i, 0)),
        out_shape=jax.ShapeDtypeStruct((2 * M, 64), jnp.float32),
    )(h, ws)


def _u_body(ea_ref, w_ref, b_ref, o_ref):
    o_ref[...] = _dot(ea_ref[...], w_ref[...]) + b_ref[0]


def _u_flat(ea, w, b):
    E = ea.shape[0]
    RB = 4096
    ws = jnp.concatenate([w[:, :64], w[:, 64:]], axis=0)
    bs = jnp.stack([b[:64], b[64:]], axis=0).reshape(2, 1, 64)
    return pl.pallas_call(
        _u_body,
        grid=(2, E // RB),
        in_specs=[
            pl.BlockSpec((RB, DB), lambda hh, i: (i, 0)),
            pl.BlockSpec((DB, 64), lambda hh, i: (hh, 0)),
            pl.BlockSpec((1, 1, 64), lambda hh, i: (hh, 0, 0)),
        ],
        out_specs=pl.BlockSpec((RB, 64), lambda hh, i: (hh * (E // RB) + i, 0)),
        out_shape=jax.ShapeDtypeStruct((2 * E, 64), jnp.float32),
    )(ea, ws, bs)


def _upd_body(h_ref, alo_ref, ahi_ref, w1_ref, w2a_ref, w2b_ref, b_ref, o_ref):
    y = (_dot(h_ref[...], w1_ref[...]) + _dot(alo_ref[...], w2a_ref[...])
         + _dot(ahi_ref[...], w2b_ref[...]) + b_ref[...])
    o_ref[...] = jnp.maximum(y, 0.0)


def _update(h, agg, wu, bu):
    M = h.shape[0]
    RB = 2048
    nb = M // RB
    return pl.pallas_call(
        _upd_body,
        grid=(nb,),
        in_specs=[
            pl.BlockSpec((RB, H), lambda i: (i, 0)),
            pl.BlockSpec((RB, 64), lambda i: (i, 0)),
            pl.BlockSpec((RB, 64), lambda i: (nb + i, 0)),
            pl.BlockSpec((H, H), lambda i: (0, 0)),
            pl.BlockSpec((64, H), lambda i: (0, 0)),
            pl.BlockSpec((64, H), lambda i: (0, 0)),
            pl.BlockSpec((1, H), lambda i: (0, 0)),
        ],
        out_specs=pl.BlockSpec((RB, H), lambda i: (i, 0)),
        out_shape=jax.ShapeDtypeStruct((M, H), jnp.float32),
    )(h, agg, agg, wu[:H], wu[H:H + 64], wu[H + 64:], bu.reshape(1, H))


def _updw_body(h_ref, alo_ref, ahi_ref, w1_ref, w2a_ref, w2b_ref, b_ref,
               w_ref, o_ref):
    y = (_dot(h_ref[...], w1_ref[...]) + _dot(alo_ref[...], w2a_ref[...])
         + _dot(ahi_ref[...], w2b_ref[...]) + b_ref[0])
    o_ref[...] = jnp.maximum(y, 0.0) * w_ref[...]


def _update_weighted(h, agg, wu, bu, fw):
    M = h.shape[0]
    RB = 2048
    nb = M // RB
    w1 = wu[:H]
    w2a = wu[H:H + 64]
    w2b = wu[H + 64:]
    w1s = jnp.concatenate([w1[:, :64], w1[:, 64:]], axis=0)
    w2as = jnp.concatenate([w2a[:, :64], w2a[:, 64:]], axis=0)
    w2bs = jnp.concatenate([w2b[:, :64], w2b[:, 64:]], axis=0)
    bs = jnp.stack([bu[:64], bu[64:]], axis=0).reshape(2, 1, 64)
    return pl.pallas_call(
        _updw_body,
        grid=(2, nb),
        in_specs=[
            pl.BlockSpec((RB, H), lambda hh, i: (i, 0)),
            pl.BlockSpec((RB, 64), lambda hh, i: (i, 0)),
            pl.BlockSpec((RB, 64), lambda hh, i: (nb + i, 0)),
            pl.BlockSpec((H, 64), lambda hh, i: (hh, 0)),
            pl.BlockSpec((64, 64), lambda hh, i: (hh, 0)),
            pl.BlockSpec((64, 64), lambda hh, i: (hh, 0)),
            pl.BlockSpec((1, 1, 64), lambda hh, i: (hh, 0, 0)),
            pl.BlockSpec((RB, 1), lambda hh, i: (i, 0)),
        ],
        out_specs=pl.BlockSpec((RB, 64), lambda hh, i: (hh * nb + i, 0)),
        out_shape=jax.ShapeDtypeStruct((2 * M, 64), jnp.float32),
    )(h, agg, agg, w1s, w2as, w2bs, bs, fw.reshape(M, 1))


def _sc_mesh():
    return plsc.VectorSubcoreMesh(core_axis_name="c", subcore_axis_name="s",
                                  num_cores=2, num_subcores=16)
_NT = 16
_CH = 128


def _edge_aggregate(t_cat, u_cat, src, dst, n):
    e = src.shape[0]
    per_tile = e // _NT
    G = per_tile // _CH
    rows_per_tile = n // _NT

    @functools.partial(
        pl.kernel,
        out_type=jax.ShapeDtypeStruct((2 * n, 64), jnp.float32),
        mesh=_sc_mesh(),
        scratch_types=[
            pltpu.VMEM((2, _CH), jnp.int32),
            pltpu.VMEM((2, _CH), jnp.int32),
            pltpu.VMEM((2, _CH, 64), jnp.float32),
            pltpu.VMEM((_CH, 64), jnp.float32),
            pltpu.VMEM_SHARED((n, 64), jnp.float32),
            pltpu.SemaphoreType.DMA,
            pltpu.SemaphoreType.DMA,
        ],
        compiler_params=pltpu.CompilerParams(use_tc_tiling_on_sc=False),
    )
    def k(tc, uc, srcr, dstr, zer, out, sidx, didx, tbuf, ubuf, acc, g0, g1):
        c = lax.axis_index("c")
        s = lax.axis_index("s")
        r0 = s * rows_per_tile
        pltpu.sync_copy(zer.at[pl.ds(r0, rows_per_tile)],
                        acc.at[pl.ds(r0, rows_per_tile)])
        plsc.subcore_barrier()
        base = s * per_tile
        toff = c * n
        sems = (g0, g1)

        def fetch(g, b):
            e0 = base + g * _CH
            pltpu.sync_copy(srcr.at[pl.ds(e0, _CH)], sidx.at[b])
            pltpu.sync_copy(dstr.at[pl.ds(e0, _CH)], didx.at[b])
            for j in range(_CH // 16):
                sl = pl.ds(j * 16, 16)
                sidx[b, sl] = sidx[b, sl] + toff
            pltpu.async_copy(tc.at[sidx.at[b]], tbuf.at[b], sems[b])

        def consume(g, b):
            e0 = base + g * _CH
            pltpu.sync_copy(uc.at[pl.ds(c * e + e0, _CH)], ubuf)
            pltpu.make_async_copy(tc.at[sidx.at[b]], tbuf.at[b],
                                  sems[b]).wait()

            def rowbody(r, rc):
                for j in range(4):
                    sl = pl.ds(j * 16, 16)
                    tbuf[b, r, sl] = jnp.maximum(tbuf[b, r, sl] + ubuf[r, sl],
                                                 0.0)
                return rc
            lax.fori_loop(0, _CH, rowbody, 0, unroll=4)
            pltpu.sync_copy(tbuf.at[b], acc.at[didx.at[b]], add=True)

        fetch(0, 0)

        def body(gg, carry):
            g = gg * 2
            fetch(g + 1, 1)
            consume(g, 0)

            @pl.when(g + 2 < G)
            def _():
                fetch(g + 2, 0)
            consume(g + 1, 1)
            return carry

        lax.fori_loop(0, G // 2, body, 0)
        plsc.subcore_barrier()
        pltpu.sync_copy(acc.at[pl.ds(r0, rows_per_tile)],
                        out.at[pl.ds(toff + r0, rows_per_tile)])

    zer = jnp.zeros((n, 64), jnp.float32)
    return k(t_cat, u_cat, src, dst, zer)


def _pool(hw, seg, nseg):
    m = seg.shape[0]
    per_tile = m // _NT
    G = per_tile // _CH
    rows_per_tile = nseg // _NT

    @functools.partial(
        pl.kernel,
        out_type=jax.ShapeDtypeStruct((2 * nseg, 64), jnp.float32),
        mesh=_sc_mesh(),
        scratch_types=[
            pltpu.VMEM((2, _CH), jnp.int32),
            pltpu.VMEM((2, _CH, 64), jnp.float32),
            pltpu.VMEM_SHARED((nseg, 64), jnp.float32),
            pltpu.SemaphoreType.DMA,
            pltpu.SemaphoreType.DMA,
        ],
        compiler_params=pltpu.CompilerParams(use_tc_tiling_on_sc=False),
    )
    def k(hwr, segr, zer, out, didx, tbuf, acc, g0, g1):
        c = lax.axis_index("c")
        s = lax.axis_index("s")
        r0 = s * rows_per_tile
        pltpu.sync_copy(zer.at[pl.ds(r0, rows_per_tile)],
                        acc.at[pl.ds(r0, rows_per_tile)])
        plsc.subcore_barrier()
        base = s * per_tile
        sems = (g0, g1)

        def fetch(g, b):
            e0 = base + g * _CH
            pltpu.sync_copy(segr.at[pl.ds(e0, _CH)], didx.at[b])
            pltpu.async_copy(hwr.at[pl.ds(c * m + e0, _CH)], tbuf.at[b],
                             sems[b])

        def consume(g, b):
            e0 = base + g * _CH
            pltpu.make_async_copy(hwr.at[pl.ds(c * m + e0, _CH)], tbuf.at[b],
                                  sems[b]).wait()
            pltpu.sync_copy(tbuf.at[b], acc.at[didx.at[b]], add=True)

        fetch(0, 0)

        def body(gg, carry):
            g = gg * 2
            fetch(g + 1, 1)
            consume(g, 0)

            @pl.when(g + 2 < G)
            def _():
                fetch(g + 2, 0)
            consume(g + 1, 1)
            return carry

        lax.fori_loop(0, G // 2, body, 0)
        plsc.subcore_barrier()
        pltpu.sync_copy(acc.at[pl.ds(r0, rows_per_tile)],
                        out.at[pl.ds(c * nseg + r0, rows_per_tile)])

    zer = jnp.zeros((nseg, 64), jnp.float32)
    return k(hw, seg, zer)


def _gnn_embed(x, ei, ea, fw, batch, gp, num_segments):
    n = x.shape[0]
    src, dst = ei[0], ei[1]
    h = _node_encode(x, gp['Win'], gp['bin'])
    nl = len(gp['layers'])
    for li, lp in enumerate(gp['layers']):
        t = _t_flat(h, lp['Wm'][:H])
        u = _u_flat(ea, lp['Wm'][H:], lp['bm'])
        agg = _edge_aggregate(t, u, src, dst, n)
        if li < nl - 1:
            h = _update(h, agg, lp['Wu'], lp['bu'])
        else:
            hw = _update_weighted(h, agg, lp['Wu'], lp['bu'], fw)
    return _pool(hw, batch, num_segments)


def _ln_in(x):
    m = jnp.mean(x, axis=-1, keepdims=True)
    v = jnp.mean((x - m) * (x - m), axis=-1, keepdims=True)
    return (x - m) * lax.rsqrt(v + 1e-5)


def _softmax3(s0, s1, s2):
    m = jnp.maximum(jnp.maximum(s0, s1), s2)
    e0 = jnp.exp(s0 - m); e1 = jnp.exp(s1 - m); e2 = jnp.exp(s2 - m)
    d = e0 + e1 + e2
    return e0 / d, e1 / d, e2 / d


def _head_mask():
    d = lax.broadcasted_iota(jnp.int32, (H, NH), 0)
    hh = lax.broadcasted_iota(jnp.int32, (H, NH), 1)
    fwd = (d // DH == hh).astype(jnp.float32)
    d2 = lax.broadcasted_iota(jnp.int32, (NH, H), 1)
    h2 = lax.broadcasted_iota(jnp.int32, (NH, H), 0)
    bwd = (d2 // DH == h2).astype(jnp.float32)
    return fwd, bwd


def _sa_layer(a, wq, wk, wv, wo, w1, w2, mh):
    mhf, mhb = mh
    q = [_dot(x, wq) for x in a]
    k = [_dot(x, wk) for x in a]
    v = [_dot(x, wv) for x in a]
    out = []
    for i in range(3):
        s = [_dot(q[i] * k[j], mhf) * 0.25 for j in range(3)]
        p = _softmax3(*s)
        o = sum(_dot(p[j], mhb) * v[j] for j in range(3))
        x = _ln_in(a[i] + _dot(o, wo))
        x = _ln_in(x + _dot(jnp.maximum(_dot(x, w1), 0.0), w2))
        out.append(x)
    return out


def _ca_layer(y, ap, wq, wk, wv, wo, w1, w2, mh):
    mhf, mhb = mh
    q = _dot(y, wq)
    k = [_dot(x, wk) for x in ap]
    v = [_dot(x, wv) for x in ap]
    s = [_dot(q * k[j], mhf) * 0.25 for j in range(3)]
    p = _softmax3(*s)
    o = sum(_dot(p[j], mhb) * v[j] for j in range(3))
    y = _ln_in(y + _dot(o, wo))
    y = _ln_in(y + _dot(jnp.maximum(_dot(y, w1), 0.0), w2))
    return y


def _prelu(x, a):
    return jnp.where(x >= 0, x, a * x)


def _head_body(*refs):
    (main_lo, main_hi,
     e10, e11, e12, e13, e14, e15,
     e20, e21, e22, e23, e24, e25,
     fw1, fb1, fa1, fw2, fb2, fa2,
     cw1, cb1, ca_, cw2, cb2) = [r[...] for r in refs[:25]]
    attn = [r[...] for r in refs[25:25 + 48]]
    o_ref = refs[-1]
    mh = _head_mask()
    main = jnp.concatenate([main_lo, main_hi], axis=-1)

    def branch(embs, fw, fb, fa, aw, off):
        rep_t = _dot(main, fw[H:])
        a = []
        for kk in range(3):
            ek = jnp.concatenate([embs[2 * kk], embs[2 * kk + 1]], axis=-1)
            a.append(_prelu(_dot(ek, fw[:H]) + rep_t + fb, fa))
        for l in range(2):
            a = _sa_layer(a, *aw[off + 6 * l: off + 6 * l + 6], mh)
        y = main
        for l in range(2):
            y = _ca_layer(y, a, *aw[off + 12 + 6 * l: off + 18 + 6 * l], mh)
        return y

    f1 = branch([e10, e11, e12, e13, e14, e15], fw1, fb1, fa1, attn, 0)
    f2 = branch([e20, e21, e22, e23, e24, e25], fw2, fb2, fa2, attn, 24)
    z = jnp.concatenate([main, f1, f2], axis=-1)
    z = _prelu(_dot(z, cw1) + cb1, ca_)
    logit = _dot(z, cw2) + cb2
    o_ref[...] = 1.0 / (1.0 + jnp.exp(-logit))


def _head(main_emb, p1, p2, params):
    main_lo, main_hi = main_emb[:B], main_emb[B:]
    embs1 = []
    embs2 = []
    for kk in range(K):
        embs1 += [p1[:B * K].reshape(B, K, 64)[:, kk],
                  p1[B * K:].reshape(B, K, 64)[:, kk]]
        embs2 += [p2[:B * K].reshape(B, K, 64)[:, kk],
                  p2[B * K:].reshape(B, K, 64)[:, kk]]
    cls = params['cls']
    w2p = jnp.pad(cls['W2'], ((0, 0), (0, 1024 - OUT)))
    b2p = jnp.pad(cls['b2'], (0, 1024 - OUT)).reshape(1, 1024)
    args = ([main_lo, main_hi] + embs1 + embs2 +
            [params['fuse1']['W'], params['fuse1']['b'].reshape(1, H),
             params['fuse1']['a'].reshape(1, 1),
             params['fuse2']['W'], params['fuse2']['b'].reshape(1, H),
             params['fuse2']['a'].reshape(1, 1),
             cls['W1'], cls['b1'].reshape(1, 3 * H), cls['a'].reshape(1, 1),
             w2p, b2p])
    for group in ('sa1', 'ca1', 'sa2', 'ca2'):
        for lp in params[group]:
            args += [lp['Wq'], lp['Wk'], lp['Wv'], lp['Wo'], lp['W1'], lp['W2']]
    out = pl.pallas_call(
        _head_body,
        out_shape=jax.ShapeDtypeStruct((B, 1024), jnp.float32),
    )(*args)
    return out[:, :OUT]


def kernel(main_x, main_edge_index, main_edge_attr, main_fc_weight, main_batch,
           add_x, add_edge_index, add_edge_attr, add_fc_weight, add_batch,
           add2_x, add2_edge_index, add2_edge_attr, add2_fc_weight, add2_batch,
           params):
    gp = params['gnn']
    main_emb = _gnn_embed(main_x, main_edge_index, main_edge_attr,
                          main_fc_weight, main_batch, gp, B)
    p1 = _gnn_embed(add_x, add_edge_index, add_edge_attr,
                    add_fc_weight, add_batch, gp, B * K)
    p2 = _gnn_embed(add2_x, add2_edge_index, add2_edge_attr,
                    add2_fc_weight, add2_batch, gp, B * K)
    return _head(main_emb, p1, p2, params)


# speedup vs baseline: 3.1142x; 1.1121x over previous
import functools

import jax
import jax.numpy as jnp
from jax import lax
from jax.experimental import pallas as pl
from jax.experimental.pallas import tpu as pltpu
from jax.experimental.pallas import tpu_sc as plsc

B = 512; K = 3; H = 128; DB = 16; NH = 8; DH = 16; OUT = 1000
NM = 24576; EM = 196608; NA = 24576; EA = 98304

_PREC = lax.Precision.HIGHEST


def _dot(a, b):
    return jnp.dot(a, b, precision=_PREC, preferred_element_type=jnp.float32)


def _enc_body(x_ref, w_ref, b_ref, o_ref):
    o_ref[...] = jnp.maximum(_dot(x_ref[...], w_ref[...]) + b_ref[...], 0.0)


def _node_encode(x, w, b):
    M = x.shape[0]
    RB = 2048
    return pl.pallas_call(
        _enc_body,
        grid=(M // RB,),
        in_specs=[
            pl.BlockSpec((RB, x.shape[1]), lambda i: (i, 0)),
            pl.BlockSpec(w.shape, lambda i: (0, 0)),
            pl.BlockSpec((1, H), lambda i: (0, 0)),
        ],
        out_specs=pl.BlockSpec((RB, H), lambda i: (i, 0)),
        out_shape=jax.ShapeDtypeStruct((M, H), jnp.float32),
    )(x, w, b.reshape(1, H))


def _t_body(h_ref, w_ref, o_ref):
    o_ref[...] = _dot(h_ref[...], w_ref[...])


def _t_flat(h, w):
    M = h.shape[0]
    RB = 2048
    ws = jnp.concatenate([w[:, :64], w[:, 64:]], axis=0)
    return pl.pallas_call(
        _t_body,
        grid=(2, M // RB),
        in_specs=[
            pl.BlockSpec((RB, H), lambda hh, i: (i, 0)),
            pl.BlockSpec((H, 64), lambda hh, i: (hh, 0)),
        ],
        out_specs=pl.BlockSpec((RB, 64), lambda hh, i: (hh * (M // RB) + i, 0)),
        out_shape=jax.ShapeDtypeStruct((2 * M, 64), jnp.float32),
    )(h, ws)


def _u_body(ea_ref, w_ref, b_ref, o_ref):
    o_ref[...] = _dot(ea_ref[...], w_ref[...]) + b_ref[0]


def _u_flat(ea, w, b):
    E = ea.shape[0]
    RB = 4096
    ws = jnp.concatenate([w[:, :64], w[:, 64:]], axis=0)
    bs = jnp.stack([b[:64], b[64:]], axis=0).reshape(2, 1, 64)
    return pl.pallas_call(
        _u_body,
        grid=(2, E // RB),
        in_specs=[
            pl.BlockSpec((RB, DB), lambda hh, i: (i, 0)),
            pl.BlockSpec((DB, 64), lambda hh, i: (hh, 0)),
            pl.BlockSpec((1, 1, 64), lambda hh, i: (hh, 0, 0)),
        ],
        out_specs=pl.BlockSpec((RB, 64), lambda hh, i: (hh * (E // RB) + i, 0)),
        out_shape=jax.ShapeDtypeStruct((2 * E, 64), jnp.float32),
    )(ea, ws, bs)


def _upd_body(h_ref, alo_ref, ahi_ref, w1_ref, w2a_ref, w2b_ref, b_ref, o_ref):
    y = (_dot(h_ref[...], w1_ref[...]) + _dot(alo_ref[...], w2a_ref[...])
         + _dot(ahi_ref[...], w2b_ref[...]) + b_ref[...])
    o_ref[...] = jnp.maximum(y, 0.0)


def _update(h, agg, wu, bu):
    M = h.shape[0]
    RB = 2048
    nb = M // RB
    return pl.pallas_call(
        _upd_body,
        grid=(nb,),
        in_specs=[
            pl.BlockSpec((RB, H), lambda i: (i, 0)),
            pl.BlockSpec((RB, 64), lambda i: (i, 0)),
            pl.BlockSpec((RB, 64), lambda i: (nb + i, 0)),
            pl.BlockSpec((H, H), lambda i: (0, 0)),
            pl.BlockSpec((64, H), lambda i: (0, 0)),
            pl.BlockSpec((64, H), lambda i: (0, 0)),
            pl.BlockSpec((1, H), lambda i: (0, 0)),
        ],
        out_specs=pl.BlockSpec((RB, H), lambda i: (i, 0)),
        out_shape=jax.ShapeDtypeStruct((M, H), jnp.float32),
    )(h, agg, agg, wu[:H], wu[H:H + 64], wu[H + 64:], bu.reshape(1, H))


def _updw_body(h_ref, alo_ref, ahi_ref, w1_ref, w2a_ref, w2b_ref, b_ref,
               w_ref, o_ref):
    y = (_dot(h_ref[...], w1_ref[...]) + _dot(alo_ref[...], w2a_ref[...])
         + _dot(ahi_ref[...], w2b_ref[...]) + b_ref[0])
    o_ref[...] = jnp.maximum(y, 0.0) * w_ref[...]


def _update_weighted(h, agg, wu, bu, fw):
    M = h.shape[0]
    RB = 2048
    nb = M // RB
    w1 = wu[:H]
    w2a = wu[H:H + 64]
    w2b = wu[H + 64:]
    w1s = jnp.concatenate([w1[:, :64], w1[:, 64:]], axis=0)
    w2as = jnp.concatenate([w2a[:, :64], w2a[:, 64:]], axis=0)
    w2bs = jnp.concatenate([w2b[:, :64], w2b[:, 64:]], axis=0)
    bs = jnp.stack([bu[:64], bu[64:]], axis=0).reshape(2, 1, 64)
    return pl.pallas_call(
        _updw_body,
        grid=(2, nb),
        in_specs=[
            pl.BlockSpec((RB, H), lambda hh, i: (i, 0)),
            pl.BlockSpec((RB, 64), lambda hh, i: (i, 0)),
            pl.BlockSpec((RB, 64), lambda hh, i: (nb + i, 0)),
            pl.BlockSpec((H, 64), lambda hh, i: (hh, 0)),
            pl.BlockSpec((64, 64), lambda hh, i: (hh, 0)),
            pl.BlockSpec((64, 64), lambda hh, i: (hh, 0)),
            pl.BlockSpec((1, 1, 64), lambda hh, i: (hh, 0, 0)),
            pl.BlockSpec((RB, 1), lambda hh, i: (i, 0)),
        ],
        out_specs=pl.BlockSpec((RB, 64), lambda hh, i: (hh * nb + i, 0)),
        out_shape=jax.ShapeDtypeStruct((2 * M, 64), jnp.float32),
    )(h, agg, agg, w1s, w2as, w2bs, bs, fw.reshape(M, 1))


def _sc_mesh():
    return plsc.VectorSubcoreMesh(core_axis_name="c", subcore_axis_name="s",
                                  num_cores=2, num_subcores=16)
_NT = 16
_CH = 128
_NB = 3


def _edge_aggregate(t_cat, u_cat, src, dst, n):
    e = src.shape[0]
    per_tile = e // _NT
    G = per_tile // _CH
    rows_per_tile = n // _NT

    @functools.partial(
        pl.kernel,
        out_type=jax.ShapeDtypeStruct((2 * n, 64), jnp.float32),
        mesh=_sc_mesh(),
        scratch_types=[
            pltpu.VMEM((_NB, _CH), jnp.int32),
            pltpu.VMEM((_NB, _CH), jnp.int32),
            pltpu.VMEM((_NB, _CH, 64), jnp.float32),
            pltpu.VMEM_SHARED((n, 64), jnp.float32),
            [pltpu.SemaphoreType.DMA] * _NB,
            [pltpu.SemaphoreType.DMA] * _NB,
            [pltpu.SemaphoreType.DMA] * _NB,
        ],
        compiler_params=pltpu.CompilerParams(use_tc_tiling_on_sc=False),
    )
    def k(tc, uc, srcr, dstr, zer, out, sidx, didx, tbuf, acc, sa, sg, ss):
        c = lax.axis_index("c")
        s = lax.axis_index("s")
        r0 = s * rows_per_tile
        pltpu.sync_copy(zer.at[pl.ds(r0, rows_per_tile)],
                        acc.at[pl.ds(r0, rows_per_tile)])
        plsc.subcore_barrier()
        base = s * per_tile
        toff = c * n

        def fetch_a(g, b):
            e0 = base + g * _CH
            pltpu.async_copy(srcr.at[pl.ds(e0, _CH)], sidx.at[b], sa[b])
            pltpu.async_copy(dstr.at[pl.ds(e0, _CH)], didx.at[b], sa[b])
            pltpu.async_copy(uc.at[pl.ds(c * e + e0, _CH)], tbuf.at[b], sa[b])

        def fetch_b(g, b):
            e0 = base + g * _CH
            pltpu.make_async_copy(srcr.at[pl.ds(e0, _CH)], sidx.at[b],
                                  sa[b]).wait()
            pltpu.make_async_copy(srcr.at[pl.ds(e0, _CH)], didx.at[b],
                                  sa[b]).wait()
            pltpu.make_async_copy(uc.at[pl.ds(c * e + e0, _CH)], tbuf.at[b],
                                  sa[b]).wait()
            for j in range(_CH // 16):
                sl = pl.ds(j * 16, 16)
                sidx[b, sl] = sidx[b, sl] + toff
            pltpu.async_copy(tc.at[sidx.at[b]], tbuf.at[b], sg[b], add=True)

        def consume(g, b):
            pltpu.make_async_copy(tc.at[sidx.at[b]], tbuf.at[b], sg[b]).wait()

            def rowbody(r, rc):
                for j in range(4):
                    sl = pl.ds(j * 16, 16)
                    tbuf[b, r, sl] = jnp.maximum(tbuf[b, r, sl], 0.0)
                return rc
            lax.fori_loop(0, _CH, rowbody, 0, unroll=4)
            pltpu.async_copy(tbuf.at[b], acc.at[didx.at[b]], ss[b], add=True)

        def drain(b):
            pltpu.make_async_copy(tbuf.at[b], acc.at[didx.at[b]], ss[b]).wait()

        fetch_a(0, 0)
        fetch_a(1, 1)
        fetch_b(0, 0)

        def body(gg, carry):
            for j in range(_NB):
                g = gg * _NB + j

                @pl.when(g + 1 < G)
                def _():
                    fetch_b(g + 1, (j + 1) % _NB)
                consume(g, j)

                @pl.when((g + 2 < G) & (g >= 1))
                def _():
                    drain((j + 2) % _NB)

                @pl.when(g + 2 < G)
                def _():
                    fetch_a(g + 2, (j + 2) % _NB)
            return carry

        lax.fori_loop(0, G // _NB, body, 0)
        drain((G - 3) % _NB)
        drain((G - 2) % _NB)
        drain((G - 1) % _NB)
        plsc.subcore_barrier()
        pltpu.sync_copy(acc.at[pl.ds(r0, rows_per_tile)],
                        out.at[pl.ds(toff + r0, rows_per_tile)])

    zer = jnp.zeros((n, 64), jnp.float32)
    return k(t_cat, u_cat, src, dst, zer)


def _pool(hw, seg, nseg):
    m = seg.shape[0]
    per_tile = m // _NT
    G = per_tile // _CH
    rows_per_tile = nseg // _NT

    @functools.partial(
        pl.kernel,
        out_type=jax.ShapeDtypeStruct((2 * nseg, 64), jnp.float32),
        mesh=_sc_mesh(),
        scratch_types=[
            pltpu.VMEM((_NB, _CH), jnp.int32),
            pltpu.VMEM((_NB, _CH, 64), jnp.float32),
            pltpu.VMEM_SHARED((nseg, 64), jnp.float32),
            [pltpu.SemaphoreType.DMA] * _NB,
            [pltpu.SemaphoreType.DMA] * _NB,
        ],
        compiler_params=pltpu.CompilerParams(use_tc_tiling_on_sc=False),
    )
    def k(hwr, segr, zer, out, didx, tbuf, acc, sa, ss):
        c = lax.axis_index("c")
        s = lax.axis_index("s")
        r0 = s * rows_per_tile
        pltpu.sync_copy(zer.at[pl.ds(r0, rows_per_tile)],
                        acc.at[pl.ds(r0, rows_per_tile)])
        plsc.subcore_barrier()
        base = s * per_tile

        def fetch(g, b):
            e0 = base + g * _CH
            pltpu.async_copy(segr.at[pl.ds(e0, _CH)], didx.at[b], sa[b])
            pltpu.async_copy(hwr.at[pl.ds(c * m + e0, _CH)], tbuf.at[b], sa[b])

        def consume(g, b):
            e0 = base + g * _CH
            pltpu.make_async_copy(segr.at[pl.ds(e0, _CH)], didx.at[b],
                                  sa[b]).wait()
            pltpu.make_async_copy(hwr.at[pl.ds(c * m + e0, _CH)], tbuf.at[b],
                                  sa[b]).wait()
            pltpu.async_copy(tbuf.at[b], acc.at[didx.at[b]], ss[b], add=True)

        def drain(b):
            pltpu.make_async_copy(tbuf.at[b], acc.at[didx.at[b]], ss[b]).wait()

        fetch(0, 0)
        fetch(1, 1)

        def body(gg, carry):
            for j in range(_NB):
                g = gg * _NB + j
                consume(g, j)

                @pl.when((g + 2 < G) & (g >= 1))
                def _():
                    drain((j + 2) % _NB)

                @pl.when(g + 2 < G)
                def _():
                    fetch(g + 2, (j + 2) % _NB)
            return carry

        lax.fori_loop(0, G // _NB, body, 0)
        drain((G - 3) % _NB)
        drain((G - 2) % _NB)
        drain((G - 1) % _NB)
        plsc.subcore_barrier()
        pltpu.sync_copy(acc.at[pl.ds(r0, rows_per_tile)],
                        out.at[pl.ds(c * nseg + r0, rows_per_tile)])

    zer = jnp.zeros((nseg, 64), jnp.float32)
    return k(hw, seg, zer)


def _gnn_embed(x, ei, ea, fw, batch, gp, num_segments):
    n = x.shape[0]
    src, dst = ei[0], ei[1]
    h = _node_encode(x, gp['Win'], gp['bin'])
    nl = len(gp['layers'])
    for li, lp in enumerate(gp['layers']):
        t = _t_flat(h, lp['Wm'][:H])
        u = _u_flat(ea, lp['Wm'][H:], lp['bm'])
        agg = _edge_aggregate(t, u, src, dst, n)
        if li < nl - 1:
            h = _update(h, agg, lp['Wu'], lp['bu'])
        else:
            hw = _update_weighted(h, agg, lp['Wu'], lp['bu'], fw)
    return _pool(hw, batch, num_segments)


def _ln_in(x):
    m = jnp.mean(x, axis=-1, keepdims=True)
    v = jnp.mean((x - m) * (x - m), axis=-1, keepdims=True)
    return (x - m) * lax.rsqrt(v + 1e-5)


def _softmax3(s0, s1, s2):
    m = jnp.maximum(jnp.maximum(s0, s1), s2)
    e0 = jnp.exp(s0 - m); e1 = jnp.exp(s1 - m); e2 = jnp.exp(s2 - m)
    d = e0 + e1 + e2
    return e0 / d, e1 / d, e2 / d


def _head_mask():
    d = lax.broadcasted_iota(jnp.int32, (H, NH), 0)
    hh = lax.broadcasted_iota(jnp.int32, (H, NH), 1)
    fwd = (d // DH == hh).astype(jnp.float32)
    d2 = lax.broadcasted_iota(jnp.int32, (NH, H), 1)
    h2 = lax.broadcasted_iota(jnp.int32, (NH, H), 0)
    bwd = (d2 // DH == h2).astype(jnp.float32)
    return fwd, bwd


def _sa_layer(a, wq, wk, wv, wo, w1, w2, mh):
    mhf, mhb = mh
    q = [_dot(x, wq) for x in a]
    k = [_dot(x, wk) for x in a]
    v = [_dot(x, wv) for x in a]
    out = []
    for i in range(3):
        s = [_dot(q[i] * k[j], mhf) * 0.25 for j in range(3)]
        p = _softmax3(*s)
        o = sum(_dot(p[j], mhb) * v[j] for j in range(3))
        x = _ln_in(a[i] + _dot(o, wo))
        x = _ln_in(x + _dot(jnp.maximum(_dot(x, w1), 0.0), w2))
        out.append(x)
    return out


def _ca_layer(y, ap, wq, wk, wv, wo, w1, w2, mh):
    mhf, mhb = mh
    q = _dot(y, wq)
    k = [_dot(x, wk) for x in ap]
    v = [_dot(x, wv) for x in ap]
    s = [_dot(q * k[j], mhf) * 0.25 for j in range(3)]
    p = _softmax3(*s)
    o = sum(_dot(p[j], mhb) * v[j] for j in range(3))
    y = _ln_in(y + _dot(o, wo))
    y = _ln_in(y + _dot(jnp.maximum(_dot(y, w1), 0.0), w2))
    return y


def _prelu(x, a):
    return jnp.where(x >= 0, x, a * x)


def _head_body(*refs):
    (main_lo, main_hi,
     e10, e11, e12, e13, e14, e15,
     e20, e21, e22, e23, e24, e25,
     fw1, fb1, fa1, fw2, fb2, fa2,
     cw1, cb1, ca_, cw2, cb2) = [r[...] for r in refs[:25]]
    attn = [r[...] for r in refs[25:25 + 48]]
    o_ref = refs[-1]
    mh = _head_mask()
    main = jnp.concatenate([main_lo, main_hi], axis=-1)

    def branch(embs, fw, fb, fa, aw, off):
        rep_t = _dot(main, fw[H:])
        a = []
        for kk in range(3):
            ek = jnp.concatenate([embs[2 * kk], embs[2 * kk + 1]], axis=-1)
            a.append(_prelu(_dot(ek, fw[:H]) + rep_t + fb, fa))
        for l in range(2):
            a = _sa_layer(a, *aw[off + 6 * l: off + 6 * l + 6], mh)
        y = main
        for l in range(2):
            y = _ca_layer(y, a, *aw[off + 12 + 6 * l: off + 18 + 6 * l], mh)
        return y

    f1 = branch([e10, e11, e12, e13, e14, e15], fw1, fb1, fa1, attn, 0)
    f2 = branch([e20, e21, e22, e23, e24, e25], fw2, fb2, fa2, attn, 24)
    z = jnp.concatenate([main, f1, f2], axis=-1)
    z = _prelu(_dot(z, cw1) + cb1, ca_)
    logit = _dot(z, cw2) + cb2
    o_ref[...] = 1.0 / (1.0 + jnp.exp(-logit))


def _head(main_emb, p1, p2, params):
    main_lo, main_hi = main_emb[:B], main_emb[B:]
    embs1 = []
    embs2 = []
    for kk in range(K):
        embs1 += [p1[:B * K].reshape(B, K, 64)[:, kk],
                  p1[B * K:].reshape(B, K, 64)[:, kk]]
        embs2 += [p2[:B * K].reshape(B, K, 64)[:, kk],
                  p2[B * K:].reshape(B, K, 64)[:, kk]]
    cls = params['cls']
    w2p = jnp.pad(cls['W2'], ((0, 0), (0, 1024 - OUT)))
    b2p = jnp.pad(cls['b2'], (0, 1024 - OUT)).reshape(1, 1024)
    args = ([main_lo, main_hi] + embs1 + embs2 +
            [params['fuse1']['W'], params['fuse1']['b'].reshape(1, H),
             params['fuse1']['a'].reshape(1, 1),
             params['fuse2']['W'], params['fuse2']['b'].reshape(1, H),
             params['fuse2']['a'].reshape(1, 1),
             cls['W1'], cls['b1'].reshape(1, 3 * H), cls['a'].reshape(1, 1),
             w2p, b2p])
    for group in ('sa1', 'ca1', 'sa2', 'ca2'):
        for lp in params[group]:
            args += [lp['Wq'], lp['Wk'], lp['Wv'], lp['Wo'], lp['W1'], lp['W2']]
    out = pl.pallas_call(
        _head_body,
        out_shape=jax.ShapeDtypeStruct((B, 1024), jnp.float32),
    )(*args)
    return out[:, :OUT]


def kernel(main_x, main_edge_index, main_edge_attr, main_fc_weight, main_batch,
           add_x, add_edge_index, add_edge_attr, add_fc_weight, add_batch,
           add2_x, add2_edge_index, add2_edge_attr, add2_fc_weight, add2_batch,
           params):
    gp = params['gnn']
    main_emb = _gnn_embed(main_x, main_edge_index, main_edge_attr,
                          main_fc_weight, main_batch, gp, B)
    p1 = _gnn_embed(add_x, add_edge_index, add_edge_attr,
                    add_fc_weight, add_batch, gp, B * K)
    p2 = _gnn_embed(add2_x, add2_edge_index, add2_edge_attr,
                    add2_fc_weight, add2_batch, gp, B * K)
    return _head(main_emb, p1, p2, params)
